# Initial kernel scaffold; baseline (speedup 1.0000x reference)
import jax
import jax.numpy as jnp
from jax.experimental import pallas as pl


def kernel(edge_attr1, edge_attr2):
    raise NotImplementedError("write your pallas kernel here")


# speedup vs baseline: 2.7044x; 2.7044x over previous
import functools
from math import comb

import jax
import jax.numpy as jnp
from jax import lax
from jax.experimental import pallas as pl
from jax.experimental.pallas import tpu as pltpu
from jax.experimental.pallas import tpu_sc as plsc

MAXL = 3
NR, NA, NCH = 4, 20, 2
LANES = 16
EBLK = 256
NRA = NR * NA

L_LIST = [(lx, ly, lz)
          for lx in range(MAXL + 1)
          for ly in range(MAXL + 1 - lx)
          for lz in range(MAXL + 1 - lx - ly)]
L_IDX = {l: i for i, l in enumerate(L_LIST)}

COMBOS = []
for _i1, (_x1, _y1, _z1) in enumerate(L_LIST):
    for _i2, (_x2, _y2, _z2) in enumerate(L_LIST):
        if _x1 + _x2 + _y1 + _y2 + _z1 + _z2 <= MAXL:
            _p = (comb(_x1 + _x2, _x1) * comb(_y1 + _y2, _y1)
                  * comb(_z1 + _z2, _z1))
            COMBOS.append((L_IDX[(_x1 + _x2, _y1 + _y2, _z1 + _z2)],
                           _i1, _i2, _p))
COMBOS.sort(key=lambda t: (t[2], t[1]))
assert len(COMBOS) == 84

NC, NS = 2, 16
NW = NC * NS


def _make_sc_call(NB):
    nb_lo = NB // NW
    nb_extra = NB % NW
    mesh = plsc.VectorSubcoreMesh(core_axis_name="c", subcore_axis_name="s")

    @functools.partial(
        pl.kernel,
        out_type=jax.ShapeDtypeStruct((NRA, NB, EBLK), jnp.float32),
        mesh=mesh,
        scratch_types=[
            pltpu.VMEM((NRA, EBLK), jnp.float32),
            pltpu.VMEM((NRA, EBLK), jnp.float32),
            pltpu.VMEM((NRA, EBLK), jnp.float32),
        ],
    )
    def sck(a1_hbm, a2_hbm, out_hbm, b1, b2, ob):
        wid = lax.axis_index("s") * NC + lax.axis_index("c")
        my_n = nb_lo + jnp.where(wid < nb_extra, 1, 0)
        blk0 = wid * nb_lo + lax.min(wid, nb_extra)

        def blk_body(i, carry):
            blk = blk0 + i
            pltpu.sync_copy(a1_hbm.at[:, blk, :], b1)
            pltpu.sync_copy(a2_hbm.at[:, blk, :], b2)

            def group_body(g, gcarry):
                for r in range(NR):
                    for c in range(NCH):
                        lane0 = c * 128 + g * LANES
                        a1v = [b1[r * NA + a, pl.ds(lane0, LANES)]
                               for a in range(NA)]
                        acc = {}
                        a2_cache = {}
                        for i3, i1, i2, p in COMBOS:
                            if i2 not in a2_cache:
                                a2_cache[i2] = b2[r * NA + i2,
                                                  pl.ds(lane0, LANES)]
                            t = a1v[i1] * a2_cache[i2]
                            if p != 1:
                                t = t * jnp.float32(p)
                            acc[i3] = t if i3 not in acc else acc[i3] + t
                        for i3, v in acc.items():
                            ob[r * NA + i3, pl.ds(lane0, LANES)] = v
                return gcarry

            lax.fori_loop(0, 128 // LANES, group_body, 0, unroll=False)
            pltpu.sync_copy(ob, out_hbm.at[:, blk, :])
            return carry

        lax.fori_loop(0, my_n, blk_body, 0, unroll=False)

    return sck


def kernel(edge_attr1, edge_attr2):
    E = edge_attr1.shape[0]
    NB = E // 128
    def to_native(x):
        x5 = x.reshape(NB, 128, NR, NA, NCH).transpose(2, 3, 0, 4, 1)
        return x5.reshape(NRA, NB, EBLK)

    o3 = _make_sc_call(NB)(to_native(edge_attr1), to_native(edge_attr2))
    o5 = o3.reshape(NR, NA, NB, NCH, 128)
    return o5.transpose(2, 4, 0, 1, 3).reshape(E, NR, NA, NCH)


# speedup vs baseline: 12.0795x; 4.4665x over previous
import functools
from math import comb

import jax
import jax.numpy as jnp
from jax import lax
from jax.experimental import pallas as pl
from jax.experimental.pallas import tpu as pltpu
from jax.experimental.pallas import tpu_sc as plsc
from jax.experimental.layout import Layout, with_layout_constraint

MAXL = 3
NR, NA, NCH = 4, 20, 2
LANES = 16
EBLK = 256
NRA = NR * NA

L_LIST = [(lx, ly, lz)
          for lx in range(MAXL + 1)
          for ly in range(MAXL + 1 - lx)
          for lz in range(MAXL + 1 - lx - ly)]
L_IDX = {l: i for i, l in enumerate(L_LIST)}

COMBOS = []
for _i1, (_x1, _y1, _z1) in enumerate(L_LIST):
    for _i2, (_x2, _y2, _z2) in enumerate(L_LIST):
        if _x1 + _x2 + _y1 + _y2 + _z1 + _z2 <= MAXL:
            _p = (comb(_x1 + _x2, _x1) * comb(_y1 + _y2, _y1)
                  * comb(_z1 + _z2, _z1))
            COMBOS.append((L_IDX[(_x1 + _x2, _y1 + _y2, _z1 + _z2)],
                           _i1, _i2, _p))
COMBOS.sort(key=lambda t: (t[2], t[1]))
assert len(COMBOS) == 84

NC, NS = 2, 16
NW = NC * NS


def _make_sc_call(NB):
    nb_lo = NB // NW
    nb_extra = NB % NW
    mesh = plsc.VectorSubcoreMesh(core_axis_name="c", subcore_axis_name="s")

    @functools.partial(
        pl.kernel,
        out_type=jax.ShapeDtypeStruct((NRA * NB * NCH, 128), jnp.float32),
        mesh=mesh,
        scratch_types=[
            pltpu.VMEM((2, NRA // 2, NCH, 128), jnp.float32),
            pltpu.VMEM((2, NRA // 2, NCH, 128), jnp.float32),
            pltpu.VMEM((2, NRA // 2, NCH, 128), jnp.float32),
        ],
    )
    def sck(a1_hbm, a2_hbm, out_hbm, b1, b2, ob):
        a1_v = a1_hbm.reshape(NRA // 2, 2 * NB * NCH, 128)
        a2_v = a2_hbm.reshape(NRA // 2, 2 * NB * NCH, 128)
        out_v = out_hbm.reshape(NRA // 2, 2 * NB * NCH, 128)
        wid = lax.axis_index("s") * NC + lax.axis_index("c")
        my_n = nb_lo + jnp.where(wid < nb_extra, 1, 0)
        blk0 = wid * nb_lo + lax.min(wid, nb_extra)

        def blk_body(i, carry):
            blk = blk0 + i
            for par in range(2):
                d1 = par * (NB * NCH) + blk * NCH
                pltpu.sync_copy(a1_v.at[:, pl.ds(d1, NCH), :], b1.at[par])
                pltpu.sync_copy(a2_v.at[:, pl.ds(d1, NCH), :], b2.at[par])

            def group_body(g, gcarry):
                lane0 = g * LANES

                def col(buf, ra, c):
                    return buf[ra % 2, ra // 2, c, pl.ds(lane0, LANES)]

                for r in range(NR):
                    for c in range(NCH):
                        a1v = [col(b1, r * NA + a, c) for a in range(NA)]
                        acc = {}
                        a2_cache = {}
                        for i3, i1, i2, p in COMBOS:
                            if i2 not in a2_cache:
                                a2_cache[i2] = col(b2, r * NA + i2, c)
                            t = a1v[i1] * a2_cache[i2]
                            if p != 1:
                                t = t * jnp.float32(p)
                            acc[i3] = t if i3 not in acc else acc[i3] + t
                        for i3, v in acc.items():
                            ra = r * NA + i3
                            ob[ra % 2, ra // 2, c, pl.ds(lane0, LANES)] = v
                return gcarry

            lax.fori_loop(0, 128 // LANES, group_body, 0, unroll=False)
            for par in range(2):
                d1 = par * (NB * NCH) + blk * NCH
                pltpu.sync_copy(ob.at[par], out_v.at[:, pl.ds(d1, NCH), :])
            return carry

        lax.fori_loop(0, my_n, blk_body, 0, unroll=False)

    return sck


def kernel(edge_attr1, edge_attr2):
    E = edge_attr1.shape[0]
    NB = E // 128
    lay_split = Layout(major_to_minor=(2, 3, 0, 4, 1), tiling=((2, 128),))
    lay_rm5 = Layout(major_to_minor=(0, 1, 2, 3, 4), tiling=((2, 128),))

    def to_native(x):
        s1 = with_layout_constraint(
            x.reshape(NB, 128, NR, NA, NCH), lay_split)
        s2 = with_layout_constraint(s1.transpose(2, 3, 0, 4, 1), lay_rm5)
        return s2.reshape(NRA * NB * NCH, 128)

    o2 = _make_sc_call(NB)(to_native(edge_attr1), to_native(edge_attr2))
    o5 = with_layout_constraint(
        o2.reshape(NR, NA, NB, NCH, 128), lay_rm5)
    o5t = with_layout_constraint(o5.transpose(2, 4, 0, 1, 3), lay_split)
    return o5t.reshape(E, NR, NA, NCH)


# speedup vs baseline: 29.2676x; 2.4229x over previous
import functools
from math import comb

import jax
import jax.numpy as jnp
from jax import lax
from jax.experimental import pallas as pl
from jax.experimental.pallas import tpu as pltpu
from jax.experimental.pallas import tpu_sc as plsc
from jax.experimental.layout import Layout, with_layout_constraint

MAXL = 3
NR, NA, NCH = 4, 20, 2
LANES = 16
EBLK = 256
NRA = NR * NA

L_LIST = [(lx, ly, lz)
          for lx in range(MAXL + 1)
          for ly in range(MAXL + 1 - lx)
          for lz in range(MAXL + 1 - lx - ly)]
L_IDX = {l: i for i, l in enumerate(L_LIST)}

COMBOS = []
for _i1, (_x1, _y1, _z1) in enumerate(L_LIST):
    for _i2, (_x2, _y2, _z2) in enumerate(L_LIST):
        if _x1 + _x2 + _y1 + _y2 + _z1 + _z2 <= MAXL:
            _p = (comb(_x1 + _x2, _x1) * comb(_y1 + _y2, _y1)
                  * comb(_z1 + _z2, _z1))
            COMBOS.append((L_IDX[(_x1 + _x2, _y1 + _y2, _z1 + _z2)],
                           _i1, _i2, _p))
COMBOS.sort(key=lambda t: (t[2], t[1]))
assert len(COMBOS) == 84

NC, NS = 2, 16
NW = NC * NS


def _make_sc_call(NB):
    q, rr = divmod(NB, NW)
    NSTEP = q + (1 if rr else 0)
    assert NSTEP % 2 == 0, "2-deep ring assumes an even trip count"
    mesh = plsc.VectorSubcoreMesh(core_axis_name="c", subcore_axis_name="s")

    @functools.partial(
        pl.kernel,
        out_type=jax.ShapeDtypeStruct((NRA * NB * NCH, 128), jnp.float32),
        mesh=mesh,
        scratch_types=[
            pltpu.VMEM((2, 2, NRA // 2, NCH, 128), jnp.float32),
            pltpu.VMEM((2, 2, NRA // 2, NCH, 128), jnp.float32),
            pltpu.VMEM((2, 2, NRA // 2, NCH, 128), jnp.float32),
            pltpu.SemaphoreType.DMA,
            pltpu.SemaphoreType.DMA,
            pltpu.SemaphoreType.DMA,
            pltpu.SemaphoreType.DMA,
        ],
    )
    def sck(a1_hbm, a2_hbm, out_hbm, b1, b2, ob,
            sem_i0, sem_i1, sem_o0, sem_o1):
        a1_v = a1_hbm.reshape(NRA // 2, 2 * NB * NCH, 128)
        a2_v = a2_hbm.reshape(NRA // 2, 2 * NB * NCH, 128)
        out_v = out_hbm.reshape(NRA // 2, 2 * NB * NCH, 128)
        wid = lax.axis_index("s") * NC + lax.axis_index("c")
        blk0 = lax.min(wid * q + lax.min(wid, rr), NB - NSTEP)
        sem_in = (sem_i0, sem_i1)
        sem_out = (sem_o0, sem_o1)

        def issue_in(blk, buf):
            for par in range(2):
                d1 = par * (NB * NCH) + blk * NCH
                pltpu.async_copy(a1_v.at[:, pl.ds(d1, NCH), :],
                                 b1.at[buf, par], sem_in[buf])
                pltpu.async_copy(a2_v.at[:, pl.ds(d1, NCH), :],
                                 b2.at[buf, par], sem_in[buf])

        def wait_in(buf):
            for par in range(2):
                pltpu.make_async_copy(a1_v.at[:, pl.ds(0, NCH), :],
                                      b1.at[buf, par], sem_in[buf]).wait()
                pltpu.make_async_copy(a2_v.at[:, pl.ds(0, NCH), :],
                                      b2.at[buf, par], sem_in[buf]).wait()

        def issue_out(blk, buf):
            for par in range(2):
                d1 = par * (NB * NCH) + blk * NCH
                pltpu.async_copy(ob.at[buf, par],
                                 out_v.at[:, pl.ds(d1, NCH), :],
                                 sem_out[buf])

        def wait_out(buf):
            for par in range(2):
                pltpu.make_async_copy(ob.at[buf, par],
                                      out_v.at[:, pl.ds(0, NCH), :],
                                      sem_out[buf]).wait()

        def compute(buf):
            def group_body(g, gcarry):
                lane0 = g * LANES

                def col(b, ra, c):
                    return b[buf, ra % 2, ra // 2, c, pl.ds(lane0, LANES)]

                for r in range(NR):
                    for c in range(NCH):
                        a1v = [col(b1, r * NA + a, c) for a in range(NA)]
                        acc = {}
                        a2_cache = {}
                        for i3, i1, i2, p in COMBOS:
                            if i2 not in a2_cache:
                                a2_cache[i2] = col(b2, r * NA + i2, c)
                            t = a1v[i1] * a2_cache[i2]
                            if p != 1:
                                t = t * jnp.float32(p)
                            acc[i3] = t if i3 not in acc else acc[i3] + t
                        for i3, v in acc.items():
                            ra = r * NA + i3
                            ob[buf, ra % 2, ra // 2, c,
                               pl.ds(lane0, LANES)] = v
                return gcarry

            lax.fori_loop(0, 128 // LANES, group_body, 0, unroll=False)

        issue_in(blk0, 0)

        def outer_body(i2, carry):
            for buf in range(2):
                k = i2 * 2 + buf

                @pl.when(k < NSTEP - 1)
                def _():
                    issue_in(blk0 + k + 1, 1 - buf)

                wait_in(buf)

                @pl.when(k >= 2)
                def _():
                    wait_out(buf)

                compute(buf)
                issue_out(blk0 + k, buf)
            return carry

        lax.fori_loop(0, NSTEP // 2, outer_body, 0, unroll=False)
        wait_out(0)
        wait_out(1)

    return sck


def kernel(edge_attr1, edge_attr2):
    E = edge_attr1.shape[0]
    NB = E // 128
    lay_split = Layout(major_to_minor=(2, 3, 0, 4, 1), tiling=((2, 128),))
    lay_rm5 = Layout(major_to_minor=(0, 1, 2, 3, 4), tiling=((2, 128),))

    def to_native(x):
        s1 = with_layout_constraint(
            x.reshape(NB, 128, NR, NA, NCH), lay_split)
        s2 = with_layout_constraint(s1.transpose(2, 3, 0, 4, 1), lay_rm5)
        return s2.reshape(NRA * NB * NCH, 128)

    o2 = _make_sc_call(NB)(to_native(edge_attr1), to_native(edge_attr2))
    o5 = with_layout_constraint(
        o2.reshape(NR, NA, NB, NCH, 128), lay_rm5)
    o5t = with_layout_constraint(o5.transpose(2, 4, 0, 1, 3), lay_split)
    return o5t.reshape(E, NR, NA, NCH)
